# baseline (speedup 1.0000x reference)
import jax
import jax.numpy as jnp
from jax import lax
from jax.experimental import pallas as pl
from jax.experimental.pallas import tpu as pltpu
from jax.experimental.pallas import tpu_sc as plsc

_R, _C = 200, 16384
_NW = 32
_COLS_W = _C // _NW
_CCH = 128
_NCHUNK = _COLS_W // _CCH
_LANES = 16


def _sc_body(x_hbm, out_hbm, xin0, xin1, outb0, outb1,
             sin0, sin1, sout0, sout1):
    c = lax.axis_index("c")
    s = lax.axis_index("s")
    wid = s * 2 + c
    cbase = wid * _COLS_W

    xin = (xin0, xin1)
    outb = (outb0, outb1)
    sin = (sin0, sin1)
    sout = (sout0, sout1)

    def in_slice(k):
        return x_hbm.at[:, pl.ds(cbase + k * _CCH, _CCH)]

    def out_slice(k):
        return out_hbm.at[:, pl.ds(cbase + k * _CCH, _CCH)]

    pltpu.async_copy(in_slice(0), xin0, sin0)
    pltpu.async_copy(in_slice(1), xin1, sin1)

    def chunk_pair(j, carry):
        for b in range(2):
            k = j * 2 + b
            pltpu.make_async_copy(in_slice(k), xin[b], sin[b]).wait()

            @pl.when(j > 0)
            def _():
                pltpu.make_async_copy(outb[b], out_slice(k - 2),
                                      sout[b]).wait()

            src = xin[b]
            dst = outb[b]

            @plsc.parallel_loop(0, _R, step=1, unroll=4)
            def _(r):
                for col in range(0, _CCH, _LANES):
                    xv = src[r, pl.ds(col, _LANES)]
                    dst[r, pl.ds(col, _LANES)] = (1 - (xv << 1)).astype(
                        jnp.float32)

            pltpu.async_copy(outb[b], out_slice(k), sout[b])

            @pl.when(k + 2 < _NCHUNK)
            def _():
                pltpu.async_copy(in_slice(k + 2), xin[b], sin[b])

        return carry

    lax.fori_loop(0, _NCHUNK // 2, chunk_pair, 0)

    for b in range(2):
        k = _NCHUNK - 2 + b
        pltpu.make_async_copy(outb[b], out_slice(k), sout[b]).wait()


@jax.jit
def kernel(x):
    xt = x.astype(jnp.int32).T
    mesh = plsc.VectorSubcoreMesh(core_axis_name="c", subcore_axis_name="s")
    f = pl.kernel(
        _sc_body,
        out_type=jax.ShapeDtypeStruct((_R, _C), jnp.float32),
        mesh=mesh,
        scratch_types=[
            pltpu.VMEM((_R, _CCH), jnp.int32),
            pltpu.VMEM((_R, _CCH), jnp.int32),
            pltpu.VMEM((_R, _CCH), jnp.float32),
            pltpu.VMEM((_R, _CCH), jnp.float32),
            pltpu.SemaphoreType.DMA,
            pltpu.SemaphoreType.DMA,
            pltpu.SemaphoreType.DMA,
            pltpu.SemaphoreType.DMA,
        ],
    )
    return f(xt).T
